# Initial kernel scaffold; baseline (speedup 1.0000x reference)
import jax
import jax.numpy as jnp
from jax.experimental import pallas as pl


def kernel(x, edge_index, edge_weight, W0, W1, b0, b1, scale0, scale1, offset0, offset1):
    raise NotImplementedError("write your pallas kernel here")


# speedup vs baseline: 2.4606x; 2.4606x over previous
import functools

import jax
import jax.numpy as jnp
from jax import lax
from jax.experimental import pallas as pl
from jax.experimental.pallas import tpu as pltpu
from jax.experimental.pallas import tpu_sc as plsc

N = 10000
E = 320000
D = 128

NC = 2
NS = 16
NW = NC * NS
L = 16

CHUNK = 128
EDGES_PER_TILE = 10240
NCHUNK = EDGES_PER_TILE // CHUNK
EP = EDGES_PER_TILE * NW
ACC_ROWS = 10240
ROWS_PER_TILE = ACC_ROWS // NS


@functools.partial(
    pl.kernel,
    out_type=jax.ShapeDtypeStruct((NC, ACC_ROWS, D), jnp.float32),
    mesh=plsc.VectorSubcoreMesh(core_axis_name="c", subcore_axis_name="s"),
    scratch_types=[
        pltpu.VMEM((NCHUNK, CHUNK), jnp.int32),
        pltpu.VMEM((NCHUNK, CHUNK), jnp.int32),
        pltpu.VMEM((CHUNK * L,), jnp.float32),
        pltpu.VMEM((CHUNK, D), jnp.float32),
        pltpu.VMEM_SHARED((ACC_ROWS, D), jnp.float32),
        pltpu.SemaphoreType.DMA,
    ],
)
def _spmm_sc(x_hbm, src_hbm, dst_hbm, w_hbm, out_hbm,
             src_v, dst_v, w_v, rows_v, acc_sh, sem):
    c = lax.axis_index("c")
    s = lax.axis_index("s")
    wid = s * NC + c

    def zrow(i, _):
        for j in range(D // L):
            rows_v[i, pl.ds(j * L, L)] = jnp.zeros((L,), jnp.float32)
        return 0
    lax.fori_loop(0, CHUNK, zrow, 0)

    row0 = s * ROWS_PER_TILE

    def zacc(k, _):
        pltpu.sync_copy(rows_v, acc_sh.at[pl.ds(row0 + k * CHUNK, CHUNK)])
        return 0
    lax.fori_loop(0, ROWS_PER_TILE // CHUNK, zacc, 0)
    plsc.subcore_barrier()

    pltpu.sync_copy(src_hbm.at[wid], src_v)
    pltpu.sync_copy(dst_hbm.at[wid], dst_v)

    def chunk_body(g, _):
        pltpu.sync_copy(w_hbm.at[wid, g], w_v)
        pltpu.async_copy(x_hbm.at[src_v.at[g]], rows_v, sem).wait()

        def edge_body(e, _):
            wb = w_v[pl.ds(e * L, L)]
            for j in range(D // L):
                rows_v[e, pl.ds(j * L, L)] = rows_v[e, pl.ds(j * L, L)] * wb
            return 0
        lax.fori_loop(0, CHUNK, edge_body, 0)

        pltpu.sync_copy(rows_v, acc_sh.at[dst_v.at[g]], add=True)
        return 0
    lax.fori_loop(0, NCHUNK, chunk_body, 0)
    plsc.subcore_barrier()

    def wout(k, _):
        r = row0 + k * CHUNK
        pltpu.sync_copy(acc_sh.at[pl.ds(r, CHUNK)],
                        out_hbm.at[c, pl.ds(r, CHUNK)])
        return 0
    lax.fori_loop(0, ROWS_PER_TILE // CHUNK, wout, 0)


def _dense_body(x_ref, acc_ref, w0_ref, w1_ref, b0_ref, b1_ref,
                s0_ref, s1_ref, o0_ref, o1_ref, out_ref):
    h1 = acc_ref[0] + acc_ref[1]

    def trans(f, w_ref, b, sc, off):
        h = lax.dot_general(f, w_ref[...], (((1,), (1,)), ((), ())),
                            preferred_element_type=jnp.float32,
                            precision=lax.Precision.HIGHEST)
        h = jnp.maximum(h + b, 0.0)
        mean = jnp.mean(h, axis=1, keepdims=True)
        cent = h - mean
        var = jnp.mean(cent * cent, axis=1, keepdims=True) + 1e-9
        return cent * sc * lax.rsqrt(var) + off

    out_ref[...] = (trans(x_ref[...], w0_ref, b0_ref[...], s0_ref[...], o0_ref[...])
                    + trans(h1, w1_ref, b1_ref[...], s1_ref[...], o1_ref[...]))


_BLK = 512


def _dense_tc(x_p, acc, W0, W1, b0, b1, scale0, scale1, offset0, offset1):
    grid = (ACC_ROWS // _BLK,)
    vec_spec = pl.BlockSpec((1, D), lambda i: (0, 0))
    return pl.pallas_call(
        _dense_body,
        grid=grid,
        in_specs=[
            pl.BlockSpec((_BLK, D), lambda i: (i, 0)),
            pl.BlockSpec((NC, _BLK, D), lambda i: (0, i, 0)),
            pl.BlockSpec((D, D), lambda i: (0, 0)),
            pl.BlockSpec((D, D), lambda i: (0, 0)),
            vec_spec, vec_spec, vec_spec, vec_spec, vec_spec, vec_spec,
        ],
        out_specs=pl.BlockSpec((_BLK, D), lambda i: (i, 0)),
        out_shape=jax.ShapeDtypeStruct((ACC_ROWS, D), jnp.float32),
    )(x_p, acc, W0, W1, b0.reshape(1, D), b1.reshape(1, D),
      scale0.reshape(1, D), scale1.reshape(1, D),
      offset0.reshape(1, D), offset1.reshape(1, D))


def kernel(x, edge_index, edge_weight, W0, W1, b0, b1,
           scale0, scale1, offset0, offset1):
    dst = edge_index[0]
    src = edge_index[1]
    pad = EP - E
    src_p = jnp.concatenate(
        [src, jnp.zeros((pad,), jnp.int32)]).reshape(NW, NCHUNK, CHUNK)
    dst_p = jnp.concatenate(
        [dst, N + (jnp.arange(pad, dtype=jnp.int32) % 16)]
    ).reshape(NW, NCHUNK, CHUNK)
    w_p = jnp.repeat(
        jnp.concatenate([edge_weight, jnp.zeros((pad,), jnp.float32)]), L
    ).reshape(NW, NCHUNK, CHUNK * L)

    acc = _spmm_sc(x, src_p, dst_p, w_p)

    x_pad = jnp.concatenate([x, jnp.zeros((ACC_ROWS - N, D), jnp.float32)])
    out = _dense_tc(x_pad, acc, W0, W1, b0, b1,
                    scale0, scale1, offset0, offset1)
    return out[:N]


# speedup vs baseline: 2.7650x; 1.1237x over previous
import functools

import jax
import jax.numpy as jnp
from jax import lax
from jax.experimental import pallas as pl
from jax.experimental.pallas import tpu as pltpu
from jax.experimental.pallas import tpu_sc as plsc

N = 10000
E = 320000
D = 128

NC = 2
NS = 16
NW = NC * NS
L = 16

CHUNK = 64
EDGES_PER_TILE = 10240
NCHUNK = EDGES_PER_TILE // CHUNK
EP = EDGES_PER_TILE * NW
ACC_ROWS = 10240
ROWS_PER_TILE = ACC_ROWS // NS


@functools.partial(
    pl.kernel,
    out_type=jax.ShapeDtypeStruct((NC, ACC_ROWS, D), jnp.float32),
    mesh=plsc.VectorSubcoreMesh(core_axis_name="c", subcore_axis_name="s"),
    scratch_types=[
        pltpu.VMEM((EDGES_PER_TILE // 2,), jnp.int32),
        pltpu.VMEM((EDGES_PER_TILE // 2,), jnp.int32),
        pltpu.VMEM((2, CHUNK), jnp.int32),
        pltpu.VMEM((2, CHUNK), jnp.int32),
        pltpu.VMEM((2, CHUNK * L), jnp.float32),
        pltpu.VMEM((2, CHUNK, D), jnp.float32),
        pltpu.VMEM((2, CHUNK, D), jnp.float32),
        pltpu.VMEM_SHARED((ACC_ROWS, D), jnp.float32),
        pltpu.SemaphoreType.DMA,
        pltpu.SemaphoreType.DMA,
        pltpu.SemaphoreType.DMA,
        pltpu.SemaphoreType.DMA,
        pltpu.SemaphoreType.DMA,
        pltpu.SemaphoreType.DMA,
    ],
)
def _spmm_sc(x_hbm, srcp_hbm, dstp_hbm, w_hbm, out_hbm,
             srcp_v, dstp_v, gi_v, si_v, w_v, rows_v, sbuf_v, acc_sh,
             gsem0, gsem1, ssem0, ssem1, wsem0, wsem1):
    c = lax.axis_index("c")
    s = lax.axis_index("s")
    wid = s * NC + c
    gsem = (gsem0, gsem1)
    ssem = (ssem0, ssem1)
    wsem = (wsem0, wsem1)

    def zrow(i, _):
        for j in range(D // L):
            sbuf_v[0, i, pl.ds(j * L, L)] = jnp.zeros((L,), jnp.float32)
        return 0
    lax.fori_loop(0, CHUNK, zrow, 0)

    row0 = s * ROWS_PER_TILE

    def zacc(k, _):
        pltpu.sync_copy(sbuf_v.at[0], acc_sh.at[pl.ds(row0 + k * CHUNK, CHUNK)])
        return 0
    lax.fori_loop(0, ROWS_PER_TILE // CHUNK, zacc, 0)
    plsc.subcore_barrier()

    pltpu.sync_copy(srcp_hbm.at[wid], srcp_v)
    pltpu.sync_copy(dstp_hbm.at[wid], dstp_v)

    def build_idx(packed_ref, out_ring, b, g):
        base = g * (CHUNK // 2)
        for k in range(CHUNK // 32):
            v = packed_ref[pl.ds(base + k * 16, 16)]
            out_ring[b, pl.ds(k * 32, 16)] = v & 0xFFFF
            out_ring[b, pl.ds(k * 32 + 16, 16)] = v >> 16

    def start_gather(g, b):
        build_idx(srcp_v, gi_v, b, g)
        pltpu.async_copy(w_hbm.at[wid, g], w_v.at[b], wsem[b])
        pltpu.async_copy(x_hbm.at[gi_v.at[b]], rows_v.at[b], gsem[b])

    def wait_gather(g, b):
        pltpu.make_async_copy(w_hbm.at[wid, g], w_v.at[b], wsem[b]).wait()
        pltpu.make_async_copy(
            x_hbm.at[gi_v.at[b]], rows_v.at[b], gsem[b]).wait()

    start_gather(0, 0)
    start_gather(1, 1)

    def pair_body(p, _):
        for b in range(2):
            g = 2 * p + b
            wait_gather(g, b)

            @pl.when(p > 0)
            def _():
                pltpu.make_async_copy(
                    sbuf_v.at[b], acc_sh.at[si_v.at[b]], ssem[b]).wait()

            build_idx(dstp_v, si_v, b, g)

            @plsc.parallel_loop(0, CHUNK, step=1, unroll=4)
            def _(e):
                wb = w_v[b, pl.ds(e * L, L)]
                for j in range(D // L):
                    sbuf_v[b, e, pl.ds(j * L, L)] = (
                        rows_v[b, e, pl.ds(j * L, L)] * wb)

            pltpu.async_copy(
                sbuf_v.at[b], acc_sh.at[si_v.at[b]], ssem[b], add=True)

            @pl.when(p < NCHUNK // 2 - 1)
            def _():
                start_gather(g + 2, b)
        return 0
    lax.fori_loop(0, NCHUNK // 2, pair_body, 0)

    for b in range(2):
        pltpu.make_async_copy(
            sbuf_v.at[b], acc_sh.at[si_v.at[b]], ssem[b]).wait()
    plsc.subcore_barrier()

    def wout(k, _):
        r = row0 + k * CHUNK
        pltpu.sync_copy(acc_sh.at[pl.ds(r, CHUNK)],
                        out_hbm.at[c, pl.ds(r, CHUNK)])
        return 0
    lax.fori_loop(0, ROWS_PER_TILE // CHUNK, wout, 0)


def _dense_body(x_ref, acc_ref, w0_ref, w1_ref, b0_ref, b1_ref,
                s0_ref, s1_ref, o0_ref, o1_ref, out_ref):
    h1 = acc_ref[0] + acc_ref[1]

    def trans(f, w_ref, b, sc, off):
        h = lax.dot_general(f, w_ref[...], (((1,), (1,)), ((), ())),
                            preferred_element_type=jnp.float32,
                            precision=lax.Precision.HIGHEST)
        h = jnp.maximum(h + b, 0.0)
        mean = jnp.mean(h, axis=1, keepdims=True)
        cent = h - mean
        var = jnp.mean(cent * cent, axis=1, keepdims=True) + 1e-9
        return cent * sc * lax.rsqrt(var) + off

    out_ref[...] = (trans(x_ref[...], w0_ref, b0_ref[...], s0_ref[...], o0_ref[...])
                    + trans(h1, w1_ref, b1_ref[...], s1_ref[...], o1_ref[...]))


_BLK = 512


def _dense_tc(x_p, acc, W0, W1, b0, b1, scale0, scale1, offset0, offset1):
    grid = (ACC_ROWS // _BLK,)
    vec_spec = pl.BlockSpec((1, D), lambda i: (0, 0))
    return pl.pallas_call(
        _dense_body,
        grid=grid,
        in_specs=[
            pl.BlockSpec((_BLK, D), lambda i: (i, 0)),
            pl.BlockSpec((NC, _BLK, D), lambda i: (0, i, 0)),
            pl.BlockSpec((D, D), lambda i: (0, 0)),
            pl.BlockSpec((D, D), lambda i: (0, 0)),
            vec_spec, vec_spec, vec_spec, vec_spec, vec_spec, vec_spec,
        ],
        out_specs=pl.BlockSpec((_BLK, D), lambda i: (i, 0)),
        out_shape=jax.ShapeDtypeStruct((ACC_ROWS, D), jnp.float32),
    )(x_p, acc, W0, W1, b0.reshape(1, D), b1.reshape(1, D),
      scale0.reshape(1, D), scale1.reshape(1, D),
      offset0.reshape(1, D), offset1.reshape(1, D))


def kernel(x, edge_index, edge_weight, W0, W1, b0, b1,
           scale0, scale1, offset0, offset1):
    dst = edge_index[0]
    src = edge_index[1]
    pad = EP - E

    def pack(a):
        a4 = a.reshape(NW, EP // NW // 32, 2, 16)
        return (a4[:, :, 0, :] | (a4[:, :, 1, :] << 16)).reshape(NW, -1)

    src_p = pack(jnp.concatenate([src, jnp.zeros((pad,), jnp.int32)]))
    dst_p = pack(jnp.concatenate(
        [dst, N + (jnp.arange(pad, dtype=jnp.int32) % 16)]))
    w_p = jnp.repeat(
        jnp.concatenate([edge_weight, jnp.zeros((pad,), jnp.float32)]), L
    ).reshape(NW, NCHUNK, CHUNK * L)

    acc = _spmm_sc(x, src_p, dst_p, w_p)

    x_pad = jnp.concatenate([x, jnp.zeros((ACC_ROWS - N, D), jnp.float32)])
    out = _dense_tc(x_pad, acc, W0, W1, b0, b1,
                    scale0, scale1, offset0, offset1)
    return out[:N]


# speedup vs baseline: 2.7820x; 1.0062x over previous
import functools

import jax
import jax.numpy as jnp
from jax import lax
from jax.experimental import pallas as pl
from jax.experimental.pallas import tpu as pltpu
from jax.experimental.pallas import tpu_sc as plsc

N = 10000
E = 320000
D = 128

NC = 2
NS = 16
NW = NC * NS
L = 16

CHUNK = 64
EDGES_PER_TILE = 10240
NCHUNK = EDGES_PER_TILE // CHUNK
NPAIR = NCHUNK // 2
EP = EDGES_PER_TILE * NW
ACC_ROWS = 10240
ROWS_PER_TILE = ACC_ROWS // NS
ZCH = ROWS_PER_TILE // CHUNK


@functools.partial(
    pl.kernel,
    out_type=jax.ShapeDtypeStruct((NC, ACC_ROWS, D), jnp.float32),
    mesh=plsc.VectorSubcoreMesh(core_axis_name="c", subcore_axis_name="s"),
    scratch_types=[
        pltpu.VMEM((EDGES_PER_TILE // 2,), jnp.int32),
        pltpu.VMEM((EDGES_PER_TILE // 2,), jnp.int32),
        pltpu.VMEM((2, CHUNK), jnp.int32),
        pltpu.VMEM((2, CHUNK), jnp.int32),
        pltpu.VMEM((2, 2 * CHUNK * L), jnp.float32),
        pltpu.VMEM((2, CHUNK, D), jnp.float32),
        pltpu.VMEM((2, CHUNK, D), jnp.float32),
        pltpu.VMEM_SHARED((ACC_ROWS, D), jnp.float32),
        pltpu.SemaphoreType.DMA,
        pltpu.SemaphoreType.DMA,
        pltpu.SemaphoreType.DMA,
        pltpu.SemaphoreType.DMA,
        pltpu.SemaphoreType.DMA,
        pltpu.SemaphoreType.DMA,
    ],
    compiler_params=pltpu.CompilerParams(needs_layout_passes=False),
)
def _spmm_sc(x_hbm, srcp_hbm, dstp_hbm, w_hbm, out_hbm,
             srcp_v, dstp_v, gi_v, si_v, w_v, rows_v, sbuf_v, acc_sh,
             gsem0, gsem1, ssem0, ssem1, wsem0, wsem1):
    c = lax.axis_index("c")
    s = lax.axis_index("s")
    wid = s * NC + c
    gsem = (gsem0, gsem1)
    ssem = (ssem0, ssem1)
    wsem = (wsem0, wsem1)

    def zrow(i, _):
        for j in range(D // L):
            sbuf_v[0, i, pl.ds(j * L, L)] = jnp.zeros((L,), jnp.float32)
        return 0
    lax.fori_loop(0, CHUNK, zrow, 0)

    row0 = s * ROWS_PER_TILE

    pltpu.async_copy(srcp_hbm.at[wid], srcp_v, gsem0)
    pltpu.async_copy(dstp_hbm.at[wid], dstp_v, gsem1)
    for k in range(ZCH):
        pltpu.async_copy(
            sbuf_v.at[0], acc_sh.at[pl.ds(row0 + k * CHUNK, CHUNK)], ssem0)
    pltpu.make_async_copy(srcp_hbm.at[wid], srcp_v, gsem0).wait()
    pltpu.make_async_copy(dstp_hbm.at[wid], dstp_v, gsem1).wait()
    for k in range(ZCH):
        pltpu.make_async_copy(
            sbuf_v.at[0], acc_sh.at[pl.ds(row0 + k * CHUNK, CHUNK)],
            ssem0).wait()
    plsc.subcore_barrier()

    def build_idx(packed_ref, out_ring, b, g):
        base = g * (CHUNK // 2)
        for k in range(CHUNK // 32):
            v = packed_ref[pl.ds(base + k * 16, 16)]
            out_ring[b, pl.ds(k * 32, 16)] = v & 0xFFFF
            out_ring[b, pl.ds(k * 32 + 16, 16)] = v >> 16

    def start_gather(g, b):
        build_idx(srcp_v, gi_v, b, g)
        pltpu.async_copy(x_hbm.at[gi_v.at[b]], rows_v.at[b], gsem[b])

    def wait_gather(g, b):
        pltpu.make_async_copy(
            x_hbm.at[gi_v.at[b]], rows_v.at[b], gsem[b]).wait()

    def start_w(p, wb):
        pltpu.async_copy(w_hbm.at[wid, p], w_v.at[wb], wsem[wb])

    def wait_w(p, wb):
        pltpu.make_async_copy(w_hbm.at[wid, p], w_v.at[wb], wsem[wb]).wait()

    start_w(0, 0)
    start_w(1, 1)
    start_gather(0, 0)
    start_gather(1, 1)

    def super_body(q, _):
        for pp in range(2):
            p = 2 * q + pp
            wait_w(p, pp)
            for b in range(2):
                g = 2 * p + b
                wait_gather(g, b)

                @pl.when(g > 1)
                def _():
                    pltpu.make_async_copy(
                        sbuf_v.at[b], acc_sh.at[si_v.at[b]], ssem[b]).wait()

                build_idx(dstp_v, si_v, b, g)

                @plsc.parallel_loop(0, CHUNK, step=1, unroll=4)
                def _(e):
                    wb = w_v[pp, pl.ds((b * CHUNK + e) * L, L)]
                    for j in range(D // L):
                        sbuf_v[b, e, pl.ds(j * L, L)] = (
                            rows_v[b, e, pl.ds(j * L, L)] * wb)

                pltpu.async_copy(
                    sbuf_v.at[b], acc_sh.at[si_v.at[b]], ssem[b], add=True)

                @pl.when(g < NCHUNK - 2)
                def _():
                    start_gather(g + 2, b)

            @pl.when(p < NPAIR - 2)
            def _():
                start_w(p + 2, pp)
        return 0
    lax.fori_loop(0, NPAIR // 2, super_body, 0)

    for b in range(2):
        pltpu.make_async_copy(
            sbuf_v.at[b], acc_sh.at[si_v.at[b]], ssem[b]).wait()
    plsc.subcore_barrier()

    for k in range(ZCH):
        r = row0 + k * CHUNK
        pltpu.async_copy(acc_sh.at[pl.ds(r, CHUNK)],
                         out_hbm.at[c, pl.ds(r, CHUNK)], ssem0)
    for k in range(ZCH):
        r = row0 + k * CHUNK
        pltpu.make_async_copy(acc_sh.at[pl.ds(r, CHUNK)],
                              out_hbm.at[c, pl.ds(r, CHUNK)], ssem0).wait()


def _dense_body(x_ref, acc_ref, w0_ref, w1_ref, b0_ref, b1_ref,
                s0_ref, s1_ref, o0_ref, o1_ref, out_ref):
    h1 = acc_ref[0] + acc_ref[1]

    def trans(f, w_ref, b, sc, off):
        h = lax.dot_general(f, w_ref[...], (((1,), (1,)), ((), ())),
                            preferred_element_type=jnp.float32,
                            precision=lax.Precision.HIGHEST)
        h = jnp.maximum(h + b, 0.0)
        mean = jnp.mean(h, axis=1, keepdims=True)
        cent = h - mean
        var = jnp.mean(cent * cent, axis=1, keepdims=True) + 1e-9
        return cent * sc * lax.rsqrt(var) + off

    out_ref[...] = (trans(x_ref[...], w0_ref, b0_ref[...], s0_ref[...], o0_ref[...])
                    + trans(h1, w1_ref, b1_ref[...], s1_ref[...], o1_ref[...]))


_BLK = 400


def _dense_tc(x, acc, W0, W1, b0, b1, scale0, scale1, offset0, offset1):
    grid = (N // _BLK,)
    vec_spec = pl.BlockSpec((1, D), lambda i: (0, 0))
    return pl.pallas_call(
        _dense_body,
        grid=grid,
        in_specs=[
            pl.BlockSpec((_BLK, D), lambda i: (i, 0)),
            pl.BlockSpec((NC, _BLK, D), lambda i: (0, i, 0)),
            pl.BlockSpec((D, D), lambda i: (0, 0)),
            pl.BlockSpec((D, D), lambda i: (0, 0)),
            vec_spec, vec_spec, vec_spec, vec_spec, vec_spec, vec_spec,
        ],
        out_specs=pl.BlockSpec((_BLK, D), lambda i: (i, 0)),
        out_shape=jax.ShapeDtypeStruct((N, D), jnp.float32),
    )(x, acc, W0, W1, b0.reshape(1, D), b1.reshape(1, D),
      scale0.reshape(1, D), scale1.reshape(1, D),
      offset0.reshape(1, D), offset1.reshape(1, D))


def kernel(x, edge_index, edge_weight, W0, W1, b0, b1,
           scale0, scale1, offset0, offset1):
    dst = edge_index[0]
    src = edge_index[1]
    pad = EP - E

    def pack(a):
        a4 = a.reshape(NW, EP // NW // 32, 2, 16)
        return (a4[:, :, 0, :] | (a4[:, :, 1, :] << 16)).reshape(NW, -1)

    src_p = pack(jnp.concatenate([src, jnp.zeros((pad,), jnp.int32)]))
    dst_p = pack(jnp.concatenate(
        [dst, N + (jnp.arange(pad, dtype=jnp.int32) % 16)]))
    w_p = jnp.repeat(
        jnp.concatenate([edge_weight, jnp.zeros((pad,), jnp.float32)]), L
    ).reshape(NW, NPAIR, 2 * CHUNK * L)

    acc = _spmm_sc(x, src_p, dst_p, w_p)
    return _dense_tc(x, acc, W0, W1, b0, b1,
                     scale0, scale1, offset0, offset1)
